# Initial kernel scaffold; baseline (speedup 1.0000x reference)
import jax
import jax.numpy as jnp
from jax.experimental import pallas as pl


def kernel(x, dm_values):
    raise NotImplementedError("write your pallas kernel here")


# speedup vs baseline: 1.1956x; 1.1956x over previous
import jax
import jax.numpy as jnp
from jax import lax
from jax.experimental import pallas as pl
from jax.experimental.pallas import tpu as pltpu
from jax.experimental.pallas import tpu_sc as plsc

_N_FREQ = 1024
_N_TIME = 2048
_CHUNK = 128
_NCHUNK = _N_FREQ // _CHUNK

_NC = 2
_NS = 16
_NW = _NC * _NS

_BLK = 128


def _dispersion_curve():
    freq_indices = jnp.linspace(0.0, 1.0, _N_FREQ)
    freq_ghz = 1.0 + freq_indices * 0.5
    d = 1.0 / freq_ghz ** 2 - 1.0 / jnp.max(freq_ghz) ** 2
    d = d / (jnp.max(d) + 1e-08)
    return d * (_N_TIME * 0.2)


def _dedisperse_sc(x_rows, shifts, batch, n_pol, n_dm):
    rows_out = batch * n_dm * n_pol * _N_TIME * _NCHUNK
    rows_per_w = rows_out // _NW
    nblk = rows_per_w // _BLK
    rows_per_page = _N_TIME * _NCHUNK
    n_shift = batch * n_dm * _NCHUNK

    mesh = plsc.VectorSubcoreMesh(core_axis_name="c", subcore_axis_name="s")

    def body(x_hbm, sh_hbm, out_hbm, sh_v, idx_v, buf_v, sem):
        wid = lax.axis_index("s") * _NC + lax.axis_index("c")
        pltpu.sync_copy(sh_hbm, sh_v)
        g_base = wid * rows_per_w
        page = g_base // rows_per_page
        p_ = lax.rem(page, n_pol)
        bd = page // n_pol
        b_ = bd // n_dm
        in_base = (b_ * n_pol + p_) * rows_per_page
        s_base = bd * _NCHUNK
        t_base = lax.rem(g_base, rows_per_page) // _NCHUNK

        in_base_v = jnp.full((16,), in_base, jnp.int32)
        s_base_v = jnp.full((16,), s_base, jnp.int32)

        def step(k, carry):
            g0 = g_base + k * _BLK
            t0_v = jnp.full((16,), t_base + k * (_BLK // _NCHUNK), jnp.int32)
            for j in range(_BLK // 16):
                lane = lax.iota(jnp.int32, 16) + jnp.full((16,), j * 16, jnp.int32)
                c = lax.rem(lane, jnp.full((16,), _NCHUNK, jnp.int32))
                t = t0_v + lax.div(lane, jnp.full((16,), _NCHUNK, jnp.int32))
                s = plsc.load_gather(sh_v, [s_base_v + c])
                tin = lax.rem(t + s, jnp.full((16,), _N_TIME, jnp.int32))
                row = in_base_v + tin * jnp.full((16,), _NCHUNK, jnp.int32) + c
                idx_v[pl.ds(j * 16, 16)] = row
            pltpu.async_copy(x_hbm.at[idx_v], buf_v, sem).wait()
            pltpu.sync_copy(buf_v, out_hbm.at[pl.ds(g0, _BLK)])
            return carry

        lax.fori_loop(0, nblk, step, 0)

    f = pl.kernel(
        body,
        out_type=jax.ShapeDtypeStruct((rows_out, _CHUNK), jnp.float32),
        mesh=mesh,
        compiler_params=pltpu.CompilerParams(needs_layout_passes=False),
        scratch_types=[
            pltpu.VMEM((n_shift,), jnp.int32),
            pltpu.VMEM((_BLK,), jnp.int32),
            pltpu.VMEM((_BLK, _CHUNK), jnp.float32),
            pltpu.SemaphoreType.DMA,
        ],
    )
    return f(x_rows, shifts)


def kernel(x, dm_values):
    batch, n_pol, n_time, n_freq = x.shape
    n_dm = dm_values.shape[1]
    disp = _dispersion_curve()
    delays = dm_values[:, :, None] * disp[None, None, :]

    shifts = []
    for b in range(batch):
        for d in range(n_dm):
            sample_delays = delays[b, d]
            for fs in range(0, n_freq, _CHUNK):
                avg = sample_delays[fs:fs + _CHUNK].mean().astype(jnp.int32)
                eff = jnp.where(avg > 0, avg, 0)
                shifts.append(lax.rem(eff, jnp.int32(n_time)))
    shifts = jnp.stack(shifts)

    x_rows = x.reshape(batch * n_pol * n_time * _NCHUNK, _CHUNK)
    out_rows = _dedisperse_sc(x_rows, shifts, batch, n_pol, n_dm)
    out = out_rows.reshape(batch, n_dm, n_pol, n_time, n_freq)
    return (out, delays)


# speedup vs baseline: 1.3670x; 1.1434x over previous
import jax
import jax.numpy as jnp
from jax import lax
from jax.experimental import pallas as pl
from jax.experimental.pallas import tpu as pltpu
from jax.experimental.pallas import tpu_sc as plsc

_N_FREQ = 1024
_N_TIME = 2048
_CHUNK = 128
_NCHUNK = _N_FREQ // _CHUNK

_NC = 2
_NS = 16
_NW = _NC * _NS

_BLK = 128
_NBUF = 2


def _dispersion_curve():
    freq_indices = jnp.linspace(0.0, 1.0, _N_FREQ)
    freq_ghz = 1.0 + freq_indices * 0.5
    d = 1.0 / freq_ghz ** 2 - 1.0 / jnp.max(freq_ghz) ** 2
    d = d / (jnp.max(d) + 1e-08)
    return d * (_N_TIME * 0.2)


def _splat(v):
    return jnp.full((16,), v, jnp.int32)


def _dedisperse_sc(x_rows, shifts, batch, n_pol, n_dm):
    rows_out = batch * n_dm * n_pol * _N_TIME * _NCHUNK
    rows_per_w = rows_out // _NW
    nblk = rows_per_w // _BLK
    rows_per_page = _N_TIME * _NCHUNK
    n_shift = batch * n_dm * _NCHUNK
    assert nblk % _NBUF == 0

    mesh = plsc.VectorSubcoreMesh(core_axis_name="c", subcore_axis_name="s")

    def body(x_hbm, sh_hbm, out_hbm, sh_v, idx_v, buf_v, sem_g):
        wid = lax.axis_index("s") * _NC + lax.axis_index("c")
        pltpu.sync_copy(sh_hbm, sh_v)
        g_base = wid * rows_per_w
        page = g_base // rows_per_page
        p_ = lax.rem(page, n_pol)
        bd = page // n_pol
        b_ = bd // n_dm
        in_base = (b_ * n_pol + p_) * rows_per_page
        s_base = bd * _NCHUNK
        t_base = lax.rem(g_base, rows_per_page) // _NCHUNK

        lane = lax.iota(jnp.int32, 16)
        c_pat = lax.rem(lane, _splat(_NCHUNK))
        s_vec = plsc.load_gather(sh_v, [_splat(s_base) + c_pat])
        base_c = _splat(in_base) + c_pat
        tmask = _splat(_N_TIME - 1)
        nch = _splat(_NCHUNK)
        u = []
        for j in range(_BLK // 16):
            u.append(_splat(t_base + 2 * j) + lax.div(lane, nch) + s_vec)

        def fill_idx(slot, kk):
            off = _splat(kk * 16)
            for j in range(_BLK // 16):
                tin = (u[j] + off) & tmask
                idx_v[slot, pl.ds(j * 16, 16)] = base_c + tin * nch

        def start_gather(slot, kk):
            fill_idx(slot, kk)
            pltpu.async_copy(
                x_hbm.at[idx_v.at[slot]], buf_v.at[slot], sem_g[slot])

        def wait_gather(slot):
            pltpu.make_async_copy(
                x_hbm.at[idx_v.at[slot]], buf_v.at[slot], sem_g[slot]).wait()

        def write(slot, kk):
            pltpu.sync_copy(
                buf_v.at[slot], out_hbm.at[pl.ds(g_base + kk * _BLK, _BLK)])

        start_gather(0, 0)

        def grp(k2, carry):
            k0 = k2 * 2
            start_gather(1, k0 + 1)
            wait_gather(0)
            write(0, k0)
            start_gather(0, k0 + 2)
            wait_gather(1)
            write(1, k0 + 1)
            return carry

        lax.fori_loop(0, nblk // 2 - 1, grp, 0)
        start_gather(1, nblk - 1)
        wait_gather(0)
        write(0, nblk - 2)
        wait_gather(1)
        write(1, nblk - 1)

    f = pl.kernel(
        body,
        out_type=jax.ShapeDtypeStruct((rows_out, _CHUNK), jnp.float32),
        mesh=mesh,
        compiler_params=pltpu.CompilerParams(needs_layout_passes=False),
        scratch_types=[
            pltpu.VMEM((n_shift,), jnp.int32),
            pltpu.VMEM((_NBUF, _BLK), jnp.int32),
            pltpu.VMEM((_NBUF, _BLK, _CHUNK), jnp.float32),
            [pltpu.SemaphoreType.DMA] * _NBUF,
        ],
    )
    return f(x_rows, shifts)


def kernel(x, dm_values):
    batch, n_pol, n_time, n_freq = x.shape
    n_dm = dm_values.shape[1]
    disp = _dispersion_curve()
    delays = dm_values[:, :, None] * disp[None, None, :]

    shifts = []
    for b in range(batch):
        for d in range(n_dm):
            sample_delays = delays[b, d]
            for fs in range(0, n_freq, _CHUNK):
                avg = sample_delays[fs:fs + _CHUNK].mean().astype(jnp.int32)
                eff = jnp.where(avg > 0, avg, 0)
                shifts.append(lax.rem(eff, jnp.int32(n_time)))
    shifts = jnp.stack(shifts)

    x_rows = x.reshape(batch * n_pol * n_time * _NCHUNK, _CHUNK)
    out_rows = _dedisperse_sc(x_rows, shifts, batch, n_pol, n_dm)
    out = out_rows.reshape(batch, n_dm, n_pol, n_time, n_freq)
    return (out, delays)


# speedup vs baseline: 2.7916x; 2.0421x over previous
import jax
import jax.numpy as jnp
from jax import lax
from jax.experimental import pallas as pl
from jax.experimental.pallas import tpu as pltpu
from jax.experimental.pallas import tpu_sc as plsc

_N_FREQ = 1024
_N_TIME = 2048
_CHUNK = 128
_NCHUNK = _N_FREQ // _CHUNK

_NC = 2
_NS = 16
_NW = _NC * _NS

_TB = 32
_NBUF = 2


def _dispersion_curve():
    freq_indices = jnp.linspace(0.0, 1.0, _N_FREQ)
    freq_ghz = 1.0 + freq_indices * 0.5
    d = 1.0 / freq_ghz ** 2 - 1.0 / jnp.max(freq_ghz) ** 2
    d = d / (jnp.max(d) + 1e-08)
    return d * (_N_TIME * 0.2)


def _splat(v):
    return jnp.full((16,), v, jnp.int32)


def _dedisperse_sc(x3, shifts, batch, n_pol, n_dm):
    t_rows_out = batch * n_dm * n_pol * _N_TIME
    t_per_w = t_rows_out // _NW
    nblk = t_per_w // _TB
    n_shift = batch * n_dm * _NCHUNK

    mesh = plsc.VectorSubcoreMesh(core_axis_name="c", subcore_axis_name="s")

    def body(x_hbm, sh_hbm, out_hbm, sh_v, idx_v, buf_v, sem_g):
        wid = lax.axis_index("s") * _NC + lax.axis_index("c")
        pltpu.sync_copy(sh_hbm, sh_v)
        out_base = wid * t_per_w
        page = out_base // _N_TIME
        p_ = lax.rem(page, n_pol)
        bd = page // n_pol
        b_ = bd // n_dm
        in_base = (b_ * n_pol + p_) * _N_TIME
        s_base = bd * _NCHUNK
        t_base = lax.rem(out_base, _N_TIME)

        lane = lax.iota(jnp.int32, 16)
        tmask = _splat(_N_TIME - 1)
        in_base_v = _splat(in_base)
        u = []
        for c in range(_NCHUNK):
            s_c = plsc.load_gather(sh_v, [_splat(s_base + c)])
            u.append([_splat(t_base + h * 16) + lane + s_c
                      for h in range(_TB // 16)])

        def start_gathers(slot, kk):
            off = _splat(kk * _TB)
            for c in range(_NCHUNK):
                for h in range(_TB // 16):
                    tin = (u[c][h] + off) & tmask
                    idx_v[slot, c, pl.ds(h * 16, 16)] = in_base_v + tin
            for c in range(_NCHUNK):
                pltpu.async_copy(
                    x_hbm.at[idx_v.at[slot, c], pl.ds(c * _CHUNK, _CHUNK)],
                    buf_v.at[slot].at[:, pl.ds(c * _CHUNK, _CHUNK)],
                    sem_g[slot])

        def wait_gathers(slot):
            for c in range(_NCHUNK):
                pltpu.make_async_copy(
                    x_hbm.at[idx_v.at[slot, c], pl.ds(c * _CHUNK, _CHUNK)],
                    buf_v.at[slot].at[:, pl.ds(c * _CHUNK, _CHUNK)],
                    sem_g[slot]).wait()

        def write(slot, kk):
            pltpu.sync_copy(
                buf_v.at[slot],
                out_hbm.at[pl.ds(out_base + kk * _TB, _TB)])

        start_gathers(0, 0)

        def grp(k2, carry):
            k0 = k2 * 2
            start_gathers(1, k0 + 1)
            wait_gathers(0)
            write(0, k0)
            start_gathers(0, k0 + 2)
            wait_gathers(1)
            write(1, k0 + 1)
            return carry

        lax.fori_loop(0, nblk // 2 - 1, grp, 0)
        start_gathers(1, nblk - 1)
        wait_gathers(0)
        write(0, nblk - 2)
        wait_gathers(1)
        write(1, nblk - 1)

    f = pl.kernel(
        body,
        out_type=jax.ShapeDtypeStruct((t_rows_out, _N_FREQ), jnp.float32),
        mesh=mesh,
        compiler_params=pltpu.CompilerParams(needs_layout_passes=False),
        scratch_types=[
            pltpu.VMEM((n_shift,), jnp.int32),
            pltpu.VMEM((_NBUF, _NCHUNK, _TB), jnp.int32),
            pltpu.VMEM((_NBUF, _TB, _N_FREQ), jnp.float32),
            [pltpu.SemaphoreType.DMA] * _NBUF,
        ],
    )
    return f(x3, shifts)


def kernel(x, dm_values):
    batch, n_pol, n_time, n_freq = x.shape
    n_dm = dm_values.shape[1]
    disp = _dispersion_curve()
    delays = dm_values[:, :, None] * disp[None, None, :]

    shifts = []
    for b in range(batch):
        for d in range(n_dm):
            sample_delays = delays[b, d]
            for fs in range(0, n_freq, _CHUNK):
                avg = sample_delays[fs:fs + _CHUNK].mean().astype(jnp.int32)
                eff = jnp.where(avg > 0, avg, 0)
                shifts.append(lax.rem(eff, jnp.int32(n_time)))
    shifts = jnp.stack(shifts)

    x3 = x.reshape(batch * n_pol * n_time, n_freq)
    out3 = _dedisperse_sc(x3, shifts, batch, n_pol, n_dm)
    out = out3.reshape(batch, n_dm, n_pol, n_time, n_freq)
    return (out, delays)
